# baseline (speedup 1.0000x reference)
import functools

import jax
import jax.numpy as jnp
from jax import lax
from jax.experimental import pallas as pl
from jax.experimental.pallas import tpu as pltpu
from jax.experimental.pallas import tpu_sc as plsc

_N = 10000
_E = 320000
_D = 128
_L = 3

_NC = 2
_NS = 16
_NW = _NC * _NS
_CW = 128
_GS = 16
_NG = 5
_CH = _GS * _NG
_EPW = _CH * _CW
_EP = _NW * _EPW
_TRASH = _N
_ACC_R = 10112
_SLAB = _ACC_R // _NS
_DEG_R = 10240
_DSLAB = _DEG_R // _NS

_mesh = plsc.VectorSubcoreMesh(
    core_axis_name="c", subcore_axis_name="s", num_cores=_NC, num_subcores=_NS
)


@functools.partial(
    pl.kernel,
    out_type=jax.ShapeDtypeStruct((_NC * _DEG_R,), jnp.float32),
    mesh=_mesh,
    scratch_types=[
        pltpu.VMEM((_CH, _CW), jnp.int32),
        pltpu.VMEM((_CW,), jnp.float32),
        pltpu.VMEM((_DSLAB,), jnp.float32),
        pltpu.VMEM_SHARED((_DEG_R,), jnp.float32),
        pltpu.SemaphoreType.DMA,
    ],
)
def _deg_kernel(dst_hbm, out_hbm, didx, ones, zslab, acc, sem):
    c = lax.axis_index("c")
    s = lax.axis_index("s")
    wid = s * _NC + c

    def fill_ones(i, _):
        ones[pl.ds(i * 16, 16)] = jnp.ones((16,), jnp.float32)
        return 0

    lax.fori_loop(0, _CW // 16, fill_ones, 0)

    def fill_z(i, _):
        zslab[pl.ds(i * 16, 16)] = jnp.zeros((16,), jnp.float32)
        return 0

    lax.fori_loop(0, _DSLAB // 16, fill_z, 0)

    pltpu.sync_copy(zslab, acc.at[pl.ds(s * _DSLAB, _DSLAB)])
    plsc.subcore_barrier()

    pltpu.sync_copy(dst_hbm.at[pl.ds(wid * _CH, _CH)], didx)

    def body(j, _):
        pltpu.sync_copy(ones, acc.at[didx.at[j]], add=True)
        return 0

    lax.fori_loop(0, _CH, body, 0)

    plsc.subcore_barrier()
    pltpu.sync_copy(
        acc.at[pl.ds(s * _DSLAB, _DSLAB)],
        out_hbm.at[pl.ds(c * _DEG_R + s * _DSLAB, _DSLAB)],
    )


_TOTC = _EP // _CW


def _make_seg(h0, h1):
    assert (h0 + h1) * _NS == _TOTC and h0 % _GS == 0 and h1 % _GS == 0

    @functools.partial(
        pl.kernel,
        out_type=jax.ShapeDtypeStruct((_NC, _ACC_R, _D), jnp.float32),
        mesh=_mesh,
        scratch_types=[
            pltpu.VMEM((2, _GS, _CW), jnp.int32),
            pltpu.VMEM((2, _GS, _CW), jnp.int32),
            pltpu.VMEM((2, _CW, _D), jnp.float32),
            pltpu.VMEM_SHARED((_ACC_R, _D), jnp.float32),
            [pltpu.SemaphoreType.DMA] * 4,
            [pltpu.SemaphoreType.DMA] * 2,
            pltpu.SemaphoreType.DMA,
        ],
    )
    def seg(mt_hbm, src_hbm, dst_hbm, out_hbm, sidx, didx, rows, acc,
            gsem, ssem, isem):
        c = lax.axis_index("c")
        s = lax.axis_index("s")
        base = lax.select(c == 0, s * h0, _NS * h0 + s * h1)
        ng = lax.select(c == 0, h0 // _GS, h1 // _GS)

        def fill_z(i, _):
            rows[0, i // 8, pl.ds((i % 8) * 16, 16)] = jnp.zeros(
                (16,), jnp.float32)
            return 0

        lax.fori_loop(0, _CW * 8, fill_z, 0)

        def zcp(i, _):
            pltpu.sync_copy(rows.at[0], acc.at[pl.ds(s * _SLAB + i * _CW, _CW)])
            return 0

        lax.fori_loop(0, _SLAB // _CW, zcp, 0)
        pltpu.sync_copy(
            rows.at[0, pl.ds(0, _SLAB % _CW)],
            acc.at[pl.ds(s * _SLAB + (_SLAB // _CW) * _CW, _SLAB % _CW)],
        )
        plsc.subcore_barrier()

        def i_start(g, r):
            off = base + g * _GS
            pltpu.async_copy(src_hbm.at[pl.ds(off, _GS)], sidx.at[r], isem)
            pltpu.async_copy(dst_hbm.at[pl.ds(off, _GS)], didx.at[r], isem)

        def i_wait(r):
            pltpu.make_async_copy(
                src_hbm.at[pl.ds(0, _GS)], sidx.at[r], isem).wait()
            pltpu.make_async_copy(
                dst_hbm.at[pl.ds(0, _GS)], didx.at[r], isem).wait()

        _QW = _CW // 4
        _HW = _CW // 2

        def g_start(gb, k, b):
            for q in range(4):
                pltpu.async_copy(
                    mt_hbm.at[sidx.at[gb, k, pl.ds(q * _QW, _QW)]],
                    rows.at[b, pl.ds(q * _QW, _QW)], gsem[q])

        def g_wait(b):
            for q in range(4):
                pltpu.make_async_copy(
                    mt_hbm.at[sidx.at[0, 0, pl.ds(0, _QW)]],
                    rows.at[b, pl.ds(q * _QW, _QW)], gsem[q]).wait()

        def s_start(gb, k, b):
            for u in range(2):
                pltpu.async_copy(
                    rows.at[b, pl.ds(u * _HW, _HW)],
                    acc.at[didx.at[gb, k, pl.ds(u * _HW, _HW)]],
                    ssem[u], add=True)

        def s_wait(b):
            for u in range(2):
                pltpu.make_async_copy(
                    rows.at[b, pl.ds(u * _HW, _HW)],
                    acc.at[didx.at[0, 0, pl.ds(0, _HW)]], ssem[u]).wait()

        @pl.when(ng > 0)
        def _():
            i_start(0, 0)
            i_wait(0)

        @pl.when(ng > 1)
        def _():
            i_start(1, 1)

        def group(g, _):
            gb = lax.rem(g, 2)
            g_start(gb, 0, 0)
            for k in range(_GS):
                b = k % 2
                g_wait(b)
                if k + 1 < _GS:
                    g_start(gb, k + 1, 1 - b)
                pltpu.sync_copy(rows.at[b], acc.at[didx.at[gb, k]], add=True)

            @pl.when(g + 1 < ng)
            def _():
                i_wait(1 - gb)

            @pl.when(g + 2 < ng)
            def _():
                i_start(g + 2, gb)

            return 0

        lax.fori_loop(0, ng, group, 0)

        plsc.subcore_barrier()
        pltpu.sync_copy(
            acc.at[pl.ds(s * _SLAB, _SLAB)],
            out_hbm.at[c, pl.ds(s * _SLAB, _SLAB)],
        )

    return seg


_seg_kernel = _make_seg(_CH, _CH)


_R = 1000
_G = _N // _R


def _row_spec(shape):
    return pl.BlockSpec(shape, lambda r: (r,) + (0,) * (len(shape) - 1))


def _rep_spec(shape):
    return pl.BlockSpec(shape, lambda r: (0,) * len(shape))


def _k0_body(x_ref, ew_ref, eb_ref, w0_ref, d0_ref, d1_ref, h_ref, mt_ref, dinv_ref):
    xr = x_ref[...]
    h = jnp.dot(xr, ew_ref[...], preferred_element_type=jnp.float32) + eb_ref[...]
    dinv = lax.rsqrt(1.0 + d0_ref[...] + d1_ref[...])
    h_ref[...] = h
    dinv_ref[...] = dinv
    mt_ref[...] = dinv * jnp.dot(h, w0_ref[...], preferred_element_type=jnp.float32)


_k0_call = pl.pallas_call(
    _k0_body,
    grid=(_G,),
    in_specs=[
        _row_spec((_R, _D)),
        _rep_spec((_D, _D)),
        _rep_spec((1, _D)),
        _rep_spec((_D, _D)),
        _row_spec((_R, 1)),
        _row_spec((_R, 1)),
    ],
    out_specs=[
        _row_spec((_R, _D)),
        _row_spec((_R, _D)),
        _row_spec((_R, 1)),
    ],
    out_shape=[
        jax.ShapeDtypeStruct((_N, _D), jnp.float32),
        jax.ShapeDtypeStruct((_N, _D), jnp.float32),
        jax.ShapeDtypeStruct((_N, 1), jnp.float32),
    ],
)


def _kmid_body(h_ref, pp_ref, mt_ref, dinv_ref, b_ref, wn_ref, ho_ref, mto_ref):
    dinv = dinv_ref[...]
    mt = mt_ref[...]
    h2 = h_ref[...] + dinv * (pp_ref[0] + pp_ref[1] + mt) + b_ref[...]
    ho_ref[...] = h2
    mto_ref[...] = dinv * jnp.dot(h2, wn_ref[...], preferred_element_type=jnp.float32)


_kmid_call = pl.pallas_call(
    _kmid_body,
    grid=(_G,),
    in_specs=[
        _row_spec((_R, _D)),
        pl.BlockSpec((_NC, _R, _D), lambda r: (0, r, 0)),
        _row_spec((_R, _D)),
        _row_spec((_R, 1)),
        _rep_spec((1, _D)),
        _rep_spec((_D, _D)),
    ],
    out_specs=[
        _row_spec((_R, _D)),
        _row_spec((_R, _D)),
    ],
    out_shape=[
        jax.ShapeDtypeStruct((_N, _D), jnp.float32),
        jax.ShapeDtypeStruct((_N, _D), jnp.float32),
    ],
)


def _kfin_body(h_ref, pp_ref, mt_ref, dinv_ref, b_ref, dw_ref, db_ref, out_ref):
    dinv = dinv_ref[...]
    mt = mt_ref[...]
    h3 = h_ref[...] + dinv * (pp_ref[0] + pp_ref[1] + mt) + b_ref[...]
    out_ref[...] = (
        jnp.dot(h3, dw_ref[...], preferred_element_type=jnp.float32) + db_ref[...]
    )


_kfin_call = pl.pallas_call(
    _kfin_body,
    grid=(_G,),
    in_specs=[
        _row_spec((_R, _D)),
        pl.BlockSpec((_NC, _R, _D), lambda r: (0, r, 0)),
        _row_spec((_R, _D)),
        _row_spec((_R, 1)),
        _rep_spec((1, _D)),
        _rep_spec((_D, _D)),
        _rep_spec((1, _D)),
    ],
    out_specs=_row_spec((_R, _D)),
    out_shape=jax.ShapeDtypeStruct((_N, _D), jnp.float32),
)


def kernel(x, edge_index, enc_W, enc_b, gcn_W, gcn_b, dec_W, dec_b):
    src = edge_index[0]
    dst = edge_index[1]
    perm = jnp.argsort(src)
    src = src[perm]
    dst = dst[perm]
    pad = _EP - _E
    srcp = jnp.concatenate([src, jnp.zeros((pad,), jnp.int32)]).reshape(
        _TOTC, _CW
    )
    dstp = jnp.concatenate([dst, jnp.full((pad,), _TRASH, jnp.int32)]).reshape(
        _TOTC, _CW
    )

    degp = _deg_kernel(dstp)
    d0 = degp[:_N].reshape(_N, 1)
    d1 = degp[_DEG_R:_DEG_R + _N].reshape(_N, 1)

    h, mt, dinv = _k0_call(
        x, enc_W, enc_b.reshape(1, _D), gcn_W[0], d0, d1
    )
    out = None
    for i in range(_L):
        pp = _seg_kernel(mt, srcp, dstp)
        b = gcn_b[i].reshape(1, _D)
        if i + 1 < _L:
            h, mt = _kmid_call(h, pp, mt, dinv, b, gcn_W[i + 1])
        else:
            out = _kfin_call(h, pp, mt, dinv, b, dec_W, dec_b.reshape(1, _D))
    return out


# speedup vs baseline: 2.0002x; 2.0002x over previous
import functools

import jax
import jax.numpy as jnp
from jax import lax
from jax.experimental import pallas as pl
from jax.experimental.pallas import tpu as pltpu
from jax.experimental.pallas import tpu_sc as plsc

_N = 10000
_E = 320000
_D = 128
_L = 3

_NC = 2
_NS = 16
_NW = _NC * _NS
_CW = 128
_GS = 16
_NG = 5
_CH = _GS * _NG
_EPW = _CH * _CW
_EP = _NW * _EPW
_TRASH = _N
_ACC_R = 10240
_SLAB = _ACC_R // _NS
_DEG_R = 10240
_DSLAB = _DEG_R // _NS

_mesh = plsc.VectorSubcoreMesh(
    core_axis_name="c", subcore_axis_name="s", num_cores=_NC, num_subcores=_NS
)


@functools.partial(
    pl.kernel,
    out_type=jax.ShapeDtypeStruct((_NC * _DEG_R,), jnp.float32),
    mesh=_mesh,
    scratch_types=[
        pltpu.VMEM((_CH, _CW), jnp.int32),
        pltpu.VMEM((_CW,), jnp.float32),
        pltpu.VMEM((_DSLAB,), jnp.float32),
        pltpu.VMEM_SHARED((_DEG_R,), jnp.float32),
        pltpu.SemaphoreType.DMA,
    ],
)
def _deg_kernel(dst_hbm, out_hbm, didx, ones, zslab, acc, sem):
    c = lax.axis_index("c")
    s = lax.axis_index("s")
    wid = s * _NC + c

    def fill_ones(i, _):
        ones[pl.ds(i * 16, 16)] = jnp.ones((16,), jnp.float32)
        return 0

    lax.fori_loop(0, _CW // 16, fill_ones, 0)

    def fill_z(i, _):
        zslab[pl.ds(i * 16, 16)] = jnp.zeros((16,), jnp.float32)
        return 0

    lax.fori_loop(0, _DSLAB // 16, fill_z, 0)

    pltpu.sync_copy(zslab, acc.at[pl.ds(s * _DSLAB, _DSLAB)])
    plsc.subcore_barrier()

    pltpu.sync_copy(dst_hbm.at[pl.ds(wid * _CH, _CH)], didx)

    def body(j, _):
        pltpu.sync_copy(ones, acc.at[didx.at[j]], add=True)
        return 0

    lax.fori_loop(0, _CH, body, 0)

    plsc.subcore_barrier()
    pltpu.sync_copy(
        acc.at[pl.ds(s * _DSLAB, _DSLAB)],
        out_hbm.at[pl.ds(c * _DEG_R + s * _DSLAB, _DSLAB)],
    )


_TOTC = _EP // _CW


def _make_seg(h0, h1):
    assert (h0 + h1) * _NS == _TOTC and h0 % _GS == 0 and h1 % _GS == 0

    @functools.partial(
        pl.kernel,
        out_type=jax.ShapeDtypeStruct((_NC, _ACC_R, _D), jnp.bfloat16),
        mesh=_mesh,
        scratch_types=[
            pltpu.VMEM((2, _GS, _CW), jnp.int32),
            pltpu.VMEM((2, _GS, _CW), jnp.int32),
            pltpu.VMEM((2, _CW, _D), jnp.bfloat16),
            pltpu.VMEM((_CW, _D), jnp.bfloat16),
            pltpu.VMEM_SHARED((_ACC_R, _D), jnp.bfloat16),
            [pltpu.SemaphoreType.DMA] * 2,
            pltpu.SemaphoreType.DMA,
        ],
        compiler_params=pltpu.CompilerParams(use_tc_tiling_on_sc=False),
    )
    def seg(mt_hbm, src_hbm, dst_hbm, out_hbm, sidx, didx, brows, zrows, acc,
            gsem, isem):
        c = lax.axis_index("c")
        s = lax.axis_index("s")
        base = lax.select(c == 0, s * h0, _NS * h0 + s * h1)
        ng = lax.select(c == 0, h0 // _GS, h1 // _GS)

        for zr in range(_CW):
            for zg in range(_D // 32):
                zrows[zr, pl.ds(zg * 32, 32)] = jnp.zeros(
                    (32,), jnp.bfloat16)

        def zcp(i, _):
            pltpu.sync_copy(zrows, acc.at[pl.ds(s * _SLAB + i * _CW, _CW)])
            return 0

        lax.fori_loop(0, _SLAB // _CW, zcp, 0)
        plsc.subcore_barrier()

        def i_start(g, r):
            off = base + g * _GS
            pltpu.async_copy(src_hbm.at[pl.ds(off, _GS)], sidx.at[r], isem)
            pltpu.async_copy(dst_hbm.at[pl.ds(off, _GS)], didx.at[r], isem)

        def i_wait(r):
            pltpu.make_async_copy(
                src_hbm.at[pl.ds(0, _GS)], sidx.at[r], isem).wait()
            pltpu.make_async_copy(
                dst_hbm.at[pl.ds(0, _GS)], didx.at[r], isem).wait()

        _HW = _CW // 2

        def g_start(gb, k, b):
            for q in range(2):
                pltpu.async_copy(
                    mt_hbm.at[sidx.at[gb, k, pl.ds(q * _HW, _HW)]],
                    brows.at[b, pl.ds(q * _HW, _HW)], gsem[q])

        def g_wait(b):
            for q in range(2):
                pltpu.make_async_copy(
                    mt_hbm.at[sidx.at[0, 0, pl.ds(0, _HW)]],
                    brows.at[b, pl.ds(q * _HW, _HW)], gsem[q]).wait()

        @pl.when(ng > 0)
        def _():
            i_start(0, 0)
            i_wait(0)

        @pl.when(ng > 1)
        def _():
            i_start(1, 1)

        def group(g, _):
            gb = lax.rem(g, 2)
            g_start(gb, 0, 0)
            for k in range(_GS):
                b = k % 2
                g_wait(b)
                if k + 1 < _GS:
                    g_start(gb, k + 1, 1 - b)
                pltpu.sync_copy(brows.at[b], acc.at[didx.at[gb, k]], add=True)

            @pl.when(g + 1 < ng)
            def _():
                i_wait(1 - gb)

            @pl.when(g + 2 < ng)
            def _():
                i_start(g + 2, gb)

            return 0

        lax.fori_loop(0, ng, group, 0)

        plsc.subcore_barrier()
        pltpu.sync_copy(
            acc.at[pl.ds(s * _SLAB, _SLAB)],
            out_hbm.at[c, pl.ds(s * _SLAB, _SLAB)],
        )

    return seg


_seg_kernel = _make_seg(_CH, _CH)


_R = 1000
_G = _N // _R


def _row_spec(shape):
    return pl.BlockSpec(shape, lambda r: (r,) + (0,) * (len(shape) - 1))


def _rep_spec(shape):
    return pl.BlockSpec(shape, lambda r: (0,) * len(shape))


def _k0_body(x_ref, ew_ref, eb_ref, w0_ref, d0_ref, d1_ref, h_ref, mt_ref, dinv_ref):
    xr = x_ref[...]
    h = jnp.dot(xr, ew_ref[...], preferred_element_type=jnp.float32) + eb_ref[...]
    dinv = lax.rsqrt(1.0 + d0_ref[...] + d1_ref[...])
    h_ref[...] = h
    dinv_ref[...] = dinv
    mt_ref[...] = dinv * jnp.dot(h, w0_ref[...], preferred_element_type=jnp.float32)


_k0_call = pl.pallas_call(
    _k0_body,
    grid=(_G,),
    in_specs=[
        _row_spec((_R, _D)),
        _rep_spec((_D, _D)),
        _rep_spec((1, _D)),
        _rep_spec((_D, _D)),
        _row_spec((_R, 1)),
        _row_spec((_R, 1)),
    ],
    out_specs=[
        _row_spec((_R, _D)),
        _row_spec((_R, _D)),
        _row_spec((_R, 1)),
    ],
    out_shape=[
        jax.ShapeDtypeStruct((_N, _D), jnp.float32),
        jax.ShapeDtypeStruct((_N, _D), jnp.float32),
        jax.ShapeDtypeStruct((_N, 1), jnp.float32),
    ],
)


def _kmid_body(h_ref, pp_ref, mt_ref, dinv_ref, b_ref, wn_ref, ho_ref, mto_ref):
    dinv = dinv_ref[...]
    mt = mt_ref[...]
    pps = (pp_ref[0] + pp_ref[1]).astype(jnp.float32)
    h2 = h_ref[...] + dinv * (pps + mt) + b_ref[...]
    ho_ref[...] = h2
    mto_ref[...] = dinv * jnp.dot(h2, wn_ref[...], preferred_element_type=jnp.float32)


_kmid_call = pl.pallas_call(
    _kmid_body,
    grid=(_G,),
    in_specs=[
        _row_spec((_R, _D)),
        pl.BlockSpec((_NC, _R, _D), lambda r: (0, r, 0)),
        _row_spec((_R, _D)),
        _row_spec((_R, 1)),
        _rep_spec((1, _D)),
        _rep_spec((_D, _D)),
    ],
    out_specs=[
        _row_spec((_R, _D)),
        _row_spec((_R, _D)),
    ],
    out_shape=[
        jax.ShapeDtypeStruct((_N, _D), jnp.float32),
        jax.ShapeDtypeStruct((_N, _D), jnp.float32),
    ],
)


def _kfin_body(h_ref, pp_ref, mt_ref, dinv_ref, b_ref, dw_ref, db_ref, out_ref):
    dinv = dinv_ref[...]
    mt = mt_ref[...]
    pps = (pp_ref[0] + pp_ref[1]).astype(jnp.float32)
    h3 = h_ref[...] + dinv * (pps + mt) + b_ref[...]
    out_ref[...] = (
        jnp.dot(h3, dw_ref[...], preferred_element_type=jnp.float32) + db_ref[...]
    )


_kfin_call = pl.pallas_call(
    _kfin_body,
    grid=(_G,),
    in_specs=[
        _row_spec((_R, _D)),
        pl.BlockSpec((_NC, _R, _D), lambda r: (0, r, 0)),
        _row_spec((_R, _D)),
        _row_spec((_R, 1)),
        _rep_spec((1, _D)),
        _rep_spec((_D, _D)),
        _rep_spec((1, _D)),
    ],
    out_specs=_row_spec((_R, _D)),
    out_shape=jax.ShapeDtypeStruct((_N, _D), jnp.float32),
)


def kernel(x, edge_index, enc_W, enc_b, gcn_W, gcn_b, dec_W, dec_b):
    src = edge_index[0]
    dst = edge_index[1]
    pad = _EP - _E
    srcp = jnp.concatenate([src, jnp.zeros((pad,), jnp.int32)]).reshape(
        _TOTC, _CW
    )
    dstp = jnp.concatenate([dst, jnp.full((pad,), _TRASH, jnp.int32)]).reshape(
        _TOTC, _CW
    )

    degp = _deg_kernel(dstp)
    d0 = degp[:_N].reshape(_N, 1)
    d1 = degp[_DEG_R:_DEG_R + _N].reshape(_N, 1)

    h, mt, dinv = _k0_call(
        x, enc_W, enc_b.reshape(1, _D), gcn_W[0], d0, d1
    )


    out = None
    for i in range(_L):
        pp = _seg_kernel(mt.astype(jnp.bfloat16), srcp, dstp)
        b = gcn_b[i].reshape(1, _D)
        if i + 1 < _L:
            h, mt = _kmid_call(h, pp, mt, dinv, b, gcn_W[i + 1])
        else:
            out = _kfin_call(h, pp, mt, dinv, b, dec_W, dec_b.reshape(1, _D))
    return out


# speedup vs baseline: 2.1367x; 1.0683x over previous
import functools

import jax
import jax.numpy as jnp
from jax import lax
from jax.experimental import pallas as pl
from jax.experimental.pallas import tpu as pltpu
from jax.experimental.pallas import tpu_sc as plsc

_N = 10000
_E = 320000
_D = 128
_L = 3

_NC = 2
_NS = 16
_NW = _NC * _NS
_CW = 128
_GS = 16
_NG = 5
_CH = _GS * _NG
_EPW = _CH * _CW
_EP = _NW * _EPW
_TRASH = _N
_ACC_R = 10240
_SLAB = _ACC_R // _NS
_DEG_R = 10240
_DSLAB = _DEG_R // _NS

_mesh = plsc.VectorSubcoreMesh(
    core_axis_name="c", subcore_axis_name="s", num_cores=_NC, num_subcores=_NS
)


@functools.partial(
    pl.kernel,
    out_type=jax.ShapeDtypeStruct((_NC * _DEG_R,), jnp.float32),
    mesh=_mesh,
    scratch_types=[
        pltpu.VMEM((_CH, _CW), jnp.int32),
        pltpu.VMEM((_CW,), jnp.float32),
        pltpu.VMEM((_DSLAB,), jnp.float32),
        pltpu.VMEM_SHARED((_DEG_R,), jnp.float32),
        pltpu.SemaphoreType.DMA,
    ],
)
def _deg_kernel(dst_hbm, out_hbm, didx, ones, zslab, acc, sem):
    c = lax.axis_index("c")
    s = lax.axis_index("s")
    wid = s * _NC + c

    def fill_ones(i, _):
        ones[pl.ds(i * 16, 16)] = jnp.ones((16,), jnp.float32)
        return 0

    lax.fori_loop(0, _CW // 16, fill_ones, 0)

    def fill_z(i, _):
        zslab[pl.ds(i * 16, 16)] = jnp.zeros((16,), jnp.float32)
        return 0

    lax.fori_loop(0, _DSLAB // 16, fill_z, 0)

    pltpu.sync_copy(zslab, acc.at[pl.ds(s * _DSLAB, _DSLAB)])
    plsc.subcore_barrier()

    pltpu.sync_copy(dst_hbm.at[pl.ds(wid * _CH, _CH)], didx)

    def body(j, _):
        pltpu.sync_copy(ones, acc.at[didx.at[j]], add=True)
        return 0

    lax.fori_loop(0, _CH, body, 0)

    plsc.subcore_barrier()
    pltpu.sync_copy(
        acc.at[pl.ds(s * _DSLAB, _DSLAB)],
        out_hbm.at[pl.ds(c * _DEG_R + s * _DSLAB, _DSLAB)],
    )


_TOTC = _EP // _CW


def _make_seg(h0, h1):
    assert (h0 + h1) * _NS == _TOTC and h0 % _GS == 0 and h1 % _GS == 0

    @functools.partial(
        pl.kernel,
        out_type=jax.ShapeDtypeStruct((_NC, _ACC_R, _D), jnp.bfloat16),
        mesh=_mesh,
        scratch_types=[
            pltpu.VMEM((2, _GS, _CW), jnp.int32),
            pltpu.VMEM((2, _GS, _CW), jnp.int32),
            pltpu.VMEM((4, _CW, _D), jnp.bfloat16),
            pltpu.VMEM((_CW, _D), jnp.bfloat16),
            pltpu.VMEM_SHARED((_ACC_R, _D), jnp.bfloat16),
            [pltpu.SemaphoreType.DMA] * 4,
            pltpu.SemaphoreType.DMA,
        ],
        compiler_params=pltpu.CompilerParams(use_tc_tiling_on_sc=False),
    )
    def seg(mt_hbm, src_hbm, dst_hbm, out_hbm, sidx, didx, brows, zrows, acc,
            gsem, isem):
        c = lax.axis_index("c")
        s = lax.axis_index("s")
        base = lax.select(c == 0, s * h0, _NS * h0 + s * h1)
        ng = lax.select(c == 0, h0 // _GS, h1 // _GS)

        for zr in range(_CW):
            for zg in range(_D // 32):
                zrows[zr, pl.ds(zg * 32, 32)] = jnp.zeros(
                    (32,), jnp.bfloat16)

        def zcp(i, _):
            pltpu.sync_copy(zrows, acc.at[pl.ds(s * _SLAB + i * _CW, _CW)])
            return 0

        lax.fori_loop(0, _SLAB // _CW, zcp, 0)
        plsc.subcore_barrier()

        def i_start(g, r):
            off = base + g * _GS
            pltpu.async_copy(src_hbm.at[pl.ds(off, _GS)], sidx.at[r], isem)
            pltpu.async_copy(dst_hbm.at[pl.ds(off, _GS)], didx.at[r], isem)

        def i_wait(r):
            pltpu.make_async_copy(
                src_hbm.at[pl.ds(0, _GS)], sidx.at[r], isem).wait()
            pltpu.make_async_copy(
                dst_hbm.at[pl.ds(0, _GS)], didx.at[r], isem).wait()

        def g_start(gb, k, b):
            pltpu.async_copy(mt_hbm.at[sidx.at[gb, k]], brows.at[b], gsem[b])

        def g_wait(b):
            pltpu.make_async_copy(
                mt_hbm.at[sidx.at[0, 0]], brows.at[b], gsem[b]).wait()

        @pl.when(ng > 0)
        def _():
            i_start(0, 0)
            i_wait(0)

        @pl.when(ng > 1)
        def _():
            i_start(1, 1)

        def group(g, _):
            gb = lax.rem(g, 2)
            g_start(gb, 0, 0)
            g_start(gb, 1, 1)
            g_start(gb, 2, 2)
            for k in range(_GS):
                b = k % 4
                g_wait(b)
                if k + 3 < _GS:
                    g_start(gb, k + 3, (k + 3) % 4)
                pltpu.sync_copy(brows.at[b], acc.at[didx.at[gb, k]], add=True)

            @pl.when(g + 1 < ng)
            def _():
                i_wait(1 - gb)

            @pl.when(g + 2 < ng)
            def _():
                i_start(g + 2, gb)

            return 0

        lax.fori_loop(0, ng, group, 0)

        plsc.subcore_barrier()
        pltpu.sync_copy(
            acc.at[pl.ds(s * _SLAB, _SLAB)],
            out_hbm.at[c, pl.ds(s * _SLAB, _SLAB)],
        )

    return seg


_seg_kernel = _make_seg(_CH, _CH)


_R = 1000
_G = _N // _R


def _row_spec(shape):
    return pl.BlockSpec(shape, lambda r: (r,) + (0,) * (len(shape) - 1))


def _rep_spec(shape):
    return pl.BlockSpec(shape, lambda r: (0,) * len(shape))


def _k0_body(x_ref, ew_ref, eb_ref, w0_ref, d0_ref, d1_ref, h_ref, mt_ref, dinv_ref):
    xr = x_ref[...]
    h = jnp.dot(xr, ew_ref[...], preferred_element_type=jnp.float32) + eb_ref[...]
    dinv = lax.rsqrt(1.0 + d0_ref[...] + d1_ref[...])
    h_ref[...] = h
    dinv_ref[...] = dinv
    mt_ref[...] = dinv * jnp.dot(h, w0_ref[...], preferred_element_type=jnp.float32)


_k0_call = pl.pallas_call(
    _k0_body,
    grid=(_G,),
    in_specs=[
        _row_spec((_R, _D)),
        _rep_spec((_D, _D)),
        _rep_spec((1, _D)),
        _rep_spec((_D, _D)),
        _row_spec((_R, 1)),
        _row_spec((_R, 1)),
    ],
    out_specs=[
        _row_spec((_R, _D)),
        _row_spec((_R, _D)),
        _row_spec((_R, 1)),
    ],
    out_shape=[
        jax.ShapeDtypeStruct((_N, _D), jnp.float32),
        jax.ShapeDtypeStruct((_N, _D), jnp.float32),
        jax.ShapeDtypeStruct((_N, 1), jnp.float32),
    ],
)


def _kmid_body(h_ref, pp_ref, mt_ref, dinv_ref, b_ref, wn_ref, ho_ref, mto_ref):
    dinv = dinv_ref[...]
    mt = mt_ref[...]
    pps = (pp_ref[0] + pp_ref[1]).astype(jnp.float32)
    h2 = h_ref[...] + dinv * (pps + mt) + b_ref[...]
    ho_ref[...] = h2
    mto_ref[...] = dinv * jnp.dot(h2, wn_ref[...], preferred_element_type=jnp.float32)


_kmid_call = pl.pallas_call(
    _kmid_body,
    grid=(_G,),
    in_specs=[
        _row_spec((_R, _D)),
        pl.BlockSpec((_NC, _R, _D), lambda r: (0, r, 0)),
        _row_spec((_R, _D)),
        _row_spec((_R, 1)),
        _rep_spec((1, _D)),
        _rep_spec((_D, _D)),
    ],
    out_specs=[
        _row_spec((_R, _D)),
        _row_spec((_R, _D)),
    ],
    out_shape=[
        jax.ShapeDtypeStruct((_N, _D), jnp.float32),
        jax.ShapeDtypeStruct((_N, _D), jnp.float32),
    ],
)


def _kfin_body(h_ref, pp_ref, mt_ref, dinv_ref, b_ref, dw_ref, db_ref, out_ref):
    dinv = dinv_ref[...]
    mt = mt_ref[...]
    pps = (pp_ref[0] + pp_ref[1]).astype(jnp.float32)
    h3 = h_ref[...] + dinv * (pps + mt) + b_ref[...]
    out_ref[...] = (
        jnp.dot(h3, dw_ref[...], preferred_element_type=jnp.float32) + db_ref[...]
    )


_kfin_call = pl.pallas_call(
    _kfin_body,
    grid=(_G,),
    in_specs=[
        _row_spec((_R, _D)),
        pl.BlockSpec((_NC, _R, _D), lambda r: (0, r, 0)),
        _row_spec((_R, _D)),
        _row_spec((_R, 1)),
        _rep_spec((1, _D)),
        _rep_spec((_D, _D)),
        _rep_spec((1, _D)),
    ],
    out_specs=_row_spec((_R, _D)),
    out_shape=jax.ShapeDtypeStruct((_N, _D), jnp.float32),
)


def kernel(x, edge_index, enc_W, enc_b, gcn_W, gcn_b, dec_W, dec_b):
    src = edge_index[0]
    dst = edge_index[1]
    pad = _EP - _E
    srcp = jnp.concatenate([src, jnp.zeros((pad,), jnp.int32)]).reshape(
        _TOTC, _CW
    )
    dstp = jnp.concatenate([dst, jnp.full((pad,), _TRASH, jnp.int32)]).reshape(
        _TOTC, _CW
    )

    degp = _deg_kernel(dstp)
    d0 = degp[:_N].reshape(_N, 1)
    d1 = degp[_DEG_R:_DEG_R + _N].reshape(_N, 1)

    h, mt, dinv = _k0_call(
        x, enc_W, enc_b.reshape(1, _D), gcn_W[0], d0, d1
    )


    out = None
    for i in range(_L):
        pp = _seg_kernel(mt.astype(jnp.bfloat16), srcp, dstp)
        b = gcn_b[i].reshape(1, _D)
        if i + 1 < _L:
            h, mt = _kmid_call(h, pp, mt, dinv, b, gcn_W[i + 1])
        else:
            out = _kfin_call(h, pp, mt, dinv, b, dec_W, dec_b.reshape(1, _D))
    return out


# speedup vs baseline: 2.1459x; 1.0043x over previous
import functools

import jax
import jax.numpy as jnp
from jax import lax
from jax.experimental import pallas as pl
from jax.experimental.pallas import tpu as pltpu
from jax.experimental.pallas import tpu_sc as plsc

_N = 10000
_E = 320000
_D = 128
_L = 3

_NC = 2
_NS = 16
_NW = _NC * _NS
_CW = 128
_GS = 16
_NG = 5
_CH = _GS * _NG
_EPW = _CH * _CW
_EP = _NW * _EPW
_TRASH = _N
_ACC_R = 10240
_SLAB = _ACC_R // _NS
_DEG_R = 10240
_DSLAB = _DEG_R // _NS

_mesh = plsc.VectorSubcoreMesh(
    core_axis_name="c", subcore_axis_name="s", num_cores=_NC, num_subcores=_NS
)


@functools.partial(
    pl.kernel,
    out_type=jax.ShapeDtypeStruct((_NC * _DEG_R,), jnp.float32),
    mesh=_mesh,
    scratch_types=[
        pltpu.VMEM((_CH, _CW), jnp.int32),
        pltpu.VMEM((_CW,), jnp.float32),
        pltpu.VMEM((_DSLAB,), jnp.float32),
        pltpu.VMEM_SHARED((_DEG_R,), jnp.float32),
        pltpu.SemaphoreType.DMA,
    ],
)
def _deg_kernel(dst_hbm, out_hbm, didx, ones, zslab, acc, sem):
    c = lax.axis_index("c")
    s = lax.axis_index("s")
    wid = s * _NC + c

    def fill_ones(i, _):
        ones[pl.ds(i * 16, 16)] = jnp.ones((16,), jnp.float32)
        return 0

    lax.fori_loop(0, _CW // 16, fill_ones, 0)

    def fill_z(i, _):
        zslab[pl.ds(i * 16, 16)] = jnp.zeros((16,), jnp.float32)
        return 0

    lax.fori_loop(0, _DSLAB // 16, fill_z, 0)

    pltpu.sync_copy(zslab, acc.at[pl.ds(s * _DSLAB, _DSLAB)])
    plsc.subcore_barrier()

    pltpu.sync_copy(dst_hbm.at[pl.ds(wid * _CH, _CH)], didx)

    def body(j, _):
        pltpu.sync_copy(ones, acc.at[didx.at[j]], add=True)
        return 0

    lax.fori_loop(0, _CH, body, 0)

    plsc.subcore_barrier()
    pltpu.sync_copy(
        acc.at[pl.ds(s * _DSLAB, _DSLAB)],
        out_hbm.at[pl.ds(c * _DEG_R + s * _DSLAB, _DSLAB)],
    )


_TOTC = _EP // _CW


def _make_seg(h0, h1):
    assert (h0 + h1) * _NS == _TOTC and h0 % _GS == 0 and h1 % _GS == 0

    @functools.partial(
        pl.kernel,
        out_type=jax.ShapeDtypeStruct((_NC, _ACC_R, _D), jnp.bfloat16),
        mesh=_mesh,
        scratch_types=[
            pltpu.VMEM((2, _GS, _CW), jnp.int32),
            pltpu.VMEM((2, _GS, _CW), jnp.int32),
            pltpu.VMEM((8, _CW, _D), jnp.bfloat16),
            pltpu.VMEM((_CW, _D), jnp.bfloat16),
            pltpu.VMEM_SHARED((_ACC_R, _D), jnp.bfloat16),
            [pltpu.SemaphoreType.DMA] * 8,
            pltpu.SemaphoreType.DMA,
        ],
        compiler_params=pltpu.CompilerParams(use_tc_tiling_on_sc=False),
    )
    def seg(mt_hbm, src_hbm, dst_hbm, out_hbm, sidx, didx, brows, zrows, acc,
            gsem, isem):
        c = lax.axis_index("c")
        s = lax.axis_index("s")
        base = lax.select(c == 0, s * h0, _NS * h0 + s * h1)
        ng = lax.select(c == 0, h0 // _GS, h1 // _GS)

        for zr in range(_CW):
            for zg in range(_D // 32):
                zrows[zr, pl.ds(zg * 32, 32)] = jnp.zeros(
                    (32,), jnp.bfloat16)

        def zcp(i, _):
            pltpu.sync_copy(zrows, acc.at[pl.ds(s * _SLAB + i * _CW, _CW)])
            return 0

        lax.fori_loop(0, _SLAB // _CW, zcp, 0)
        plsc.subcore_barrier()

        def i_start(g, r):
            off = base + g * _GS
            pltpu.async_copy(src_hbm.at[pl.ds(off, _GS)], sidx.at[r], isem)
            pltpu.async_copy(dst_hbm.at[pl.ds(off, _GS)], didx.at[r], isem)

        def i_wait(r):
            pltpu.make_async_copy(
                src_hbm.at[pl.ds(0, _GS)], sidx.at[r], isem).wait()
            pltpu.make_async_copy(
                dst_hbm.at[pl.ds(0, _GS)], didx.at[r], isem).wait()

        def g_start(gb, k, b):
            pltpu.async_copy(mt_hbm.at[sidx.at[gb, k]], brows.at[b], gsem[b])

        def g_wait(b):
            pltpu.make_async_copy(
                mt_hbm.at[sidx.at[0, 0]], brows.at[b], gsem[b]).wait()

        @pl.when(ng > 0)
        def _():
            i_start(0, 0)
            i_wait(0)

        @pl.when(ng > 1)
        def _():
            i_start(1, 1)

        def group(g, _):
            gb = lax.rem(g, 2)
            for kp in range(7):
                g_start(gb, kp, kp)
            for k in range(_GS):
                b = k % 8
                g_wait(b)
                if k + 7 < _GS:
                    g_start(gb, k + 7, (k + 7) % 8)
                pltpu.sync_copy(brows.at[b], acc.at[didx.at[gb, k]], add=True)

            @pl.when(g + 1 < ng)
            def _():
                i_wait(1 - gb)

            @pl.when(g + 2 < ng)
            def _():
                i_start(g + 2, gb)

            return 0

        lax.fori_loop(0, ng, group, 0)

        plsc.subcore_barrier()
        pltpu.sync_copy(
            acc.at[pl.ds(s * _SLAB, _SLAB)],
            out_hbm.at[c, pl.ds(s * _SLAB, _SLAB)],
        )

    return seg


_seg_kernel = _make_seg(_CH, _CH)


_R = 1000
_G = _N // _R


def _row_spec(shape):
    return pl.BlockSpec(shape, lambda r: (r,) + (0,) * (len(shape) - 1))


def _rep_spec(shape):
    return pl.BlockSpec(shape, lambda r: (0,) * len(shape))


def _k0_body(x_ref, ew_ref, eb_ref, w0_ref, d0_ref, d1_ref, h_ref, mt_ref, dinv_ref):
    xr = x_ref[...]
    h = jnp.dot(xr, ew_ref[...], preferred_element_type=jnp.float32) + eb_ref[...]
    dinv = lax.rsqrt(1.0 + d0_ref[...] + d1_ref[...])
    h_ref[...] = h
    dinv_ref[...] = dinv
    mt_ref[...] = dinv * jnp.dot(h, w0_ref[...], preferred_element_type=jnp.float32)


_k0_call = pl.pallas_call(
    _k0_body,
    grid=(_G,),
    in_specs=[
        _row_spec((_R, _D)),
        _rep_spec((_D, _D)),
        _rep_spec((1, _D)),
        _rep_spec((_D, _D)),
        _row_spec((_R, 1)),
        _row_spec((_R, 1)),
    ],
    out_specs=[
        _row_spec((_R, _D)),
        _row_spec((_R, _D)),
        _row_spec((_R, 1)),
    ],
    out_shape=[
        jax.ShapeDtypeStruct((_N, _D), jnp.float32),
        jax.ShapeDtypeStruct((_N, _D), jnp.float32),
        jax.ShapeDtypeStruct((_N, 1), jnp.float32),
    ],
)


def _kmid_body(h_ref, pp_ref, mt_ref, dinv_ref, b_ref, wn_ref, ho_ref, mto_ref):
    dinv = dinv_ref[...]
    mt = mt_ref[...]
    pps = (pp_ref[0] + pp_ref[1]).astype(jnp.float32)
    h2 = h_ref[...] + dinv * (pps + mt) + b_ref[...]
    ho_ref[...] = h2
    mto_ref[...] = dinv * jnp.dot(h2, wn_ref[...], preferred_element_type=jnp.float32)


_kmid_call = pl.pallas_call(
    _kmid_body,
    grid=(_G,),
    in_specs=[
        _row_spec((_R, _D)),
        pl.BlockSpec((_NC, _R, _D), lambda r: (0, r, 0)),
        _row_spec((_R, _D)),
        _row_spec((_R, 1)),
        _rep_spec((1, _D)),
        _rep_spec((_D, _D)),
    ],
    out_specs=[
        _row_spec((_R, _D)),
        _row_spec((_R, _D)),
    ],
    out_shape=[
        jax.ShapeDtypeStruct((_N, _D), jnp.float32),
        jax.ShapeDtypeStruct((_N, _D), jnp.float32),
    ],
)


def _kfin_body(h_ref, pp_ref, mt_ref, dinv_ref, b_ref, dw_ref, db_ref, out_ref):
    dinv = dinv_ref[...]
    mt = mt_ref[...]
    pps = (pp_ref[0] + pp_ref[1]).astype(jnp.float32)
    h3 = h_ref[...] + dinv * (pps + mt) + b_ref[...]
    out_ref[...] = (
        jnp.dot(h3, dw_ref[...], preferred_element_type=jnp.float32) + db_ref[...]
    )


_kfin_call = pl.pallas_call(
    _kfin_body,
    grid=(_G,),
    in_specs=[
        _row_spec((_R, _D)),
        pl.BlockSpec((_NC, _R, _D), lambda r: (0, r, 0)),
        _row_spec((_R, _D)),
        _row_spec((_R, 1)),
        _rep_spec((1, _D)),
        _rep_spec((_D, _D)),
        _rep_spec((1, _D)),
    ],
    out_specs=_row_spec((_R, _D)),
    out_shape=jax.ShapeDtypeStruct((_N, _D), jnp.float32),
)


def kernel(x, edge_index, enc_W, enc_b, gcn_W, gcn_b, dec_W, dec_b):
    src = edge_index[0]
    dst = edge_index[1]
    pad = _EP - _E
    srcp = jnp.concatenate([src, jnp.zeros((pad,), jnp.int32)]).reshape(
        _TOTC, _CW
    )
    dstp = jnp.concatenate([dst, jnp.full((pad,), _TRASH, jnp.int32)]).reshape(
        _TOTC, _CW
    )

    degp = _deg_kernel(dstp)
    d0 = degp[:_N].reshape(_N, 1)
    d1 = degp[_DEG_R:_DEG_R + _N].reshape(_N, 1)

    h, mt, dinv = _k0_call(
        x, enc_W, enc_b.reshape(1, _D), gcn_W[0], d0, d1
    )


    out = None
    for i in range(_L):
        pp = _seg_kernel(mt.astype(jnp.bfloat16), srcp, dstp)
        b = gcn_b[i].reshape(1, _D)
        if i + 1 < _L:
            h, mt = _kmid_call(h, pp, mt, dinv, b, gcn_W[i + 1])
        else:
            out = _kfin_call(h, pp, mt, dinv, b, dec_W, dec_b.reshape(1, _D))
    return out
